# baseline (speedup 1.0000x reference)
import functools

import jax
import jax.numpy as jnp
from jax import lax
from jax.experimental import pallas as pl
from jax.experimental.pallas import tpu as pltpu
from jax.experimental.pallas import tpu_sc as plsc

_LANES = 16
_GROUP = 128
_NSLOTS = 2


@functools.partial(jax.jit, static_argnames=("seq", "bblocks", "dim"))
def _sc_lookup(ids_flat, table, *, seq, bblocks, dim):
    info = plsc.get_sparse_core_info()
    ncores, nsub = info.num_cores, info.num_subcores
    assert ncores * nsub == bblocks
    d_hi = dim // 8

    mesh = plsc.VectorSubcoreMesh(core_axis_name="c", subcore_axis_name="s")

    @functools.partial(
        pl.kernel,
        out_type=jax.ShapeDtypeStruct((seq, d_hi, bblocks, 8 * _GROUP), jnp.float32),
        mesh=mesh,
        scratch_types=[
            pltpu.VMEM((_NSLOTS, _GROUP), jnp.int32),
            pltpu.VMEM((_NSLOTS, _GROUP, dim), jnp.float32),
            pltpu.VMEM((_NSLOTS, d_hi * 8 * _GROUP), jnp.float32),
            [pltpu.SemaphoreType.DMA] * _NSLOTS,
            [pltpu.SemaphoreType.DMA] * _NSLOTS,
            [pltpu.SemaphoreType.DMA] * _NSLOTS,
        ],
        compiler_params=pltpu.CompilerParams(use_tc_tiling_on_sc=False,
                                             needs_layout_passes=False),
    )
    def k(ids_hbm, table_hbm, out_hbm, idx_v, raw_v, trans_v, gsems, osems, isems):
        wid = lax.axis_index("s") * ncores + lax.axis_index("c")
        iv = lax.iota(jnp.int32, _LANES)
        t_vecs = [iv + g8 * _LANES for g8 in range(_GROUP // _LANES)]

        def idx_copy(s, slot, sem):
            return pltpu.make_async_copy(
                ids_hbm.at[pl.ds(s * (bblocks * _GROUP) + wid * _GROUP, _GROUP)],
                idx_v.at[slot], sem)

        def wb_copy(s, slot, h, sem):
            return pltpu.make_async_copy(
                trans_v.at[slot, pl.ds(h * 8 * _GROUP, 8 * _GROUP)],
                out_hbm.at[s, h, wid], sem)

        for p in range(_NSLOTS):
            idx_copy(p, p, isems[p]).start()

        def transpose_block(p):
            raw = raw_v.at[p]
            def dbody(d, carry):
                dvec = jnp.full((_LANES,), 0, jnp.int32) + d
                for g8 in range(_GROUP // _LANES):
                    vec = plsc.load_gather(raw, [t_vecs[g8], dvec])
                    trans_v[p, pl.ds(d * _GROUP + g8 * _LANES, _LANES)] = vec
                return carry
            lax.fori_loop(0, dim, dbody, 0)

        def body(i, carry):
            for p in range(_NSLOTS):
                s = i * _NSLOTS + p
                @pl.when(i >= 1)
                def _():
                    for h in range(d_hi):
                        wb_copy(s, p, h, osems[p]).wait()
                idx_copy(s, p, isems[p]).wait()
                gcp = pltpu.async_copy(
                    table_hbm.at[idx_v.at[p]], raw_v.at[p], gsems[p])
                gcp.wait()
                idx_copy(lax.rem(s + _NSLOTS, seq), p, isems[p]).start()
                transpose_block(p)
                for h in range(d_hi):
                    wb_copy(s, p, h, osems[p]).start()
            return carry

        lax.fori_loop(0, seq // _NSLOTS, body, 0)

        for p in range(_NSLOTS):
            for h in range(d_hi):
                wb_copy(0, p, h, osems[p]).wait()
            idx_copy(0, p, isems[p]).wait()

    return k(ids_flat, table)


def kernel(token_ids, embedding_matrix):
    b, s = token_ids.shape
    _, dim = embedding_matrix.shape
    bblocks = b // _GROUP
    ids_flat = token_ids.T.reshape(-1).astype(jnp.int32)
    out5 = _sc_lookup(ids_flat, embedding_matrix,
                      seq=s, bblocks=bblocks, dim=dim)
    out = (out5.reshape(s, dim // 8, bblocks, 8, _GROUP)
               .transpose(2, 4, 0, 1, 3).reshape(b, s, dim))
    return out


# speedup vs baseline: 1.0731x; 1.0731x over previous
import functools

import jax
import jax.numpy as jnp
from jax import lax
from jax.experimental import pallas as pl
from jax.experimental.pallas import tpu as pltpu
from jax.experimental.pallas import tpu_sc as plsc

_LANES = 16
_GROUP = 128
_NSLOTS = 2


@functools.partial(jax.jit, static_argnames=("seq", "bblocks", "dim"))
def _sc_lookup(ids_flat, table, *, seq, bblocks, dim):
    info = plsc.get_sparse_core_info()
    ncores, nsub = info.num_cores, info.num_subcores
    assert ncores * nsub == bblocks
    d_hi = dim // 8

    mesh = plsc.VectorSubcoreMesh(core_axis_name="c", subcore_axis_name="s")

    @functools.partial(
        pl.kernel,
        out_type=jax.ShapeDtypeStruct((seq, d_hi, bblocks, 8 * _GROUP), jnp.float32),
        mesh=mesh,
        scratch_types=[
            pltpu.VMEM((_NSLOTS, _GROUP), jnp.int32),
            pltpu.VMEM((_NSLOTS, _GROUP, dim), jnp.float32),
            pltpu.VMEM((_NSLOTS, d_hi, 8 * _GROUP), jnp.float32),
            [pltpu.SemaphoreType.DMA] * _NSLOTS,
            [pltpu.SemaphoreType.DMA] * _NSLOTS,
            [pltpu.SemaphoreType.DMA] * _NSLOTS,
        ],
        compiler_params=pltpu.CompilerParams(use_tc_tiling_on_sc=False,
                                             needs_layout_passes=False),
    )
    def k(ids_hbm, table_hbm, out_hbm, idx_v, raw_v, trans_v, gsems, osems, isems):
        wid = lax.axis_index("s") * ncores + lax.axis_index("c")
        iv = lax.iota(jnp.int32, _LANES)
        t_vecs = [iv + g8 * _LANES for g8 in range(_GROUP // _LANES)]

        def idx_copy(s, slot, sem):
            return pltpu.make_async_copy(
                ids_hbm.at[pl.ds(s * (bblocks * _GROUP) + wid * _GROUP, _GROUP)],
                idx_v.at[slot], sem)

        def gather_copy(slot):
            return pltpu.make_async_copy(
                table_hbm.at[idx_v.at[slot]], raw_v.at[slot], gsems[slot])

        def wb_copy(s, slot):
            return pltpu.make_async_copy(
                trans_v.at[slot], out_hbm.at[s, :, wid, :], osems[slot])

        def transpose_block(slot):
            raw = raw_v.at[slot]
            for dh in range(d_hi):
                for dl in range(8):
                    dvec = jnp.full((_LANES,), 8 * dh + dl, jnp.int32)
                    for g8 in range(_GROUP // _LANES):
                        vec = plsc.load_gather(raw, [t_vecs[g8], dvec])
                        trans_v[slot, dh,
                                pl.ds(dl * _GROUP + g8 * _LANES, _LANES)] = vec

        for p in range(_NSLOTS):
            idx_copy(p, p, isems[p]).start()

        def half_step(i, p):
            s_fire = i * _NSLOTS + p
            idx_copy(s_fire, p, isems[p]).wait()
            gather_copy(p).start()
            @pl.when(i + p >= 1)
            def _():
                q = 1 - p
                s_fin = s_fire - 1
                gather_copy(q).wait()
                idx_copy(lax.rem(s_fin + _NSLOTS, seq), q, isems[q]).start()
                @pl.when(s_fin >= _NSLOTS)
                def _():
                    wb_copy(0, q).wait()
                transpose_block(q)
                wb_copy(s_fin, q).start()

        def body(i, carry):
            for p in range(_NSLOTS):
                half_step(i, p)
            return carry

        lax.fori_loop(0, seq // _NSLOTS, body, 0)

        q = (seq - 1) % _NSLOTS
        gather_copy(q).wait()
        wb_copy(0, q).wait()
        transpose_block(q)
        wb_copy(seq - 1, q).start()
        for p in range(_NSLOTS):
            wb_copy(0, p).wait()
        idx_copy(0, seq % _NSLOTS, isems[seq % _NSLOTS]).wait()

    return k(ids_flat, table)


def kernel(token_ids, embedding_matrix):
    b, s = token_ids.shape
    _, dim = embedding_matrix.shape
    bblocks = b // _GROUP
    ids_flat = token_ids.T.reshape(-1).astype(jnp.int32)
    out5 = _sc_lookup(ids_flat, embedding_matrix,
                      seq=s, bblocks=bblocks, dim=dim)
    out = (out5.reshape(s, dim // 8, bblocks, 8, _GROUP)
               .transpose(2, 4, 0, 1, 3).reshape(b, s, dim))
    return out


# speedup vs baseline: 1.6392x; 1.5276x over previous
import functools

import jax
import jax.numpy as jnp
from jax import lax
from jax.experimental import pallas as pl
from jax.experimental.pallas import tpu as pltpu
from jax.experimental.pallas import tpu_sc as plsc

_IDX_LANES = 128
_NSLOTS = 2


@functools.partial(jax.jit, static_argnames=("num_rows", "dim", "chunk_rows"))
def _sc_gather(idx2d, table, *, num_rows, dim, chunk_rows):
    info = plsc.get_sparse_core_info()
    ncores, nsub = info.num_cores, info.num_subcores
    nw = ncores * nsub
    rows_pw = num_rows // nw
    n_chunks = rows_pw // chunk_rows
    n_iters = n_chunks // _NSLOTS
    chunk_elems = chunk_rows * _IDX_LANES

    mesh = plsc.VectorSubcoreMesh(core_axis_name="c", subcore_axis_name="s")

    @functools.partial(
        pl.kernel,
        out_type=jax.ShapeDtypeStruct((num_rows * _IDX_LANES, dim), jnp.float32),
        mesh=mesh,
        scratch_types=[
            pltpu.VMEM((_NSLOTS, chunk_rows, _IDX_LANES), jnp.int32),
            pltpu.VMEM((_NSLOTS * chunk_elems, dim), jnp.float32),
            [pltpu.SemaphoreType.DMA] * _NSLOTS,
            [pltpu.SemaphoreType.DMA] * _NSLOTS,
            [pltpu.SemaphoreType.DMA] * _NSLOTS,
        ],
        compiler_params=pltpu.CompilerParams(use_tc_tiling_on_sc=False),
    )
    def k(idx_hbm, table_hbm, out_hbm, idx_v, rows_v, gsems, osems, isems):
        wid = lax.axis_index("s") * ncores + lax.axis_index("c")
        row0 = wid * rows_pw

        def idx_copy(chunk_id, slot, sem):
            return pltpu.make_async_copy(
                idx_hbm.at[pl.ds(row0 + chunk_id * chunk_rows, chunk_rows), :],
                idx_v.at[slot], sem)

        def wb_copy(chunk_id, slot, sem):
            return pltpu.make_async_copy(
                rows_v.at[pl.ds(slot * chunk_elems, chunk_elems), :],
                out_hbm.at[pl.ds((row0 + chunk_id * chunk_rows) * _IDX_LANES,
                                 chunk_elems), :],
                sem)

        for p in range(_NSLOTS):
            idx_copy(p, p, isems[p]).start()

        def body(i, carry):
            for p in range(_NSLOTS):
                g = i * _NSLOTS + p
                @pl.when(i >= 1)
                def _():
                    wb_copy(g, p, osems[p]).wait()
                idx_copy(g, p, isems[p]).wait()
                copies = [
                    pltpu.async_copy(
                        table_hbm.at[idx_v.at[p, j]],
                        rows_v.at[pl.ds(p * chunk_elems + j * _IDX_LANES,
                                        _IDX_LANES)],
                        gsems[p],
                    )
                    for j in range(chunk_rows)
                ]
                for cp in copies:
                    cp.wait()
                idx_copy(lax.rem(g + _NSLOTS, n_chunks), p, isems[p]).start()
                wb_copy(g, p, osems[p]).start()
            return carry

        lax.fori_loop(0, n_iters, body, 0)

        for p in range(_NSLOTS):
            wb_copy(0, p, osems[p]).wait()
            idx_copy(0, p, isems[p]).wait()

    return k(idx2d, table)


def kernel(token_ids, embedding_matrix):
    b, s = token_ids.shape
    _, dim = embedding_matrix.shape
    total = b * s
    num_rows = total // _IDX_LANES
    idx2d = token_ids.reshape(num_rows, _IDX_LANES).astype(jnp.int32)
    out = _sc_gather(idx2d, embedding_matrix,
                     num_rows=num_rows, dim=dim, chunk_rows=4)
    return out.reshape(b, s, dim)
